# Initial kernel scaffold; baseline (speedup 1.0000x reference)
import jax
import jax.numpy as jnp
from jax.experimental import pallas as pl


def kernel(inputs, W_head, b_head, g_head, be_head, Wb, bb, gb, beb, W_f, b_f, g_f, be_f, W_p1, b_p1, g_p1, be_p1, W_p2, b_p2, g_p2, be_p2, W_p3, b_p3):
    raise NotImplementedError("write your pallas kernel here")


# speedup vs baseline: 25.6002x; 25.6002x over previous
import functools

import jax
import jax.numpy as jnp
from jax import lax
from jax.experimental import pallas as pl
from jax.experimental.pallas import tpu as pltpu
from jax.experimental.pallas import tpu_sc as plsc

B = 4
N = 4096
KNN = 16
R = B * N
D = 64
BN_EPS = 1e-5

_KT = 256
_MT = 512
_NW = 32
_NPW = R // _NW
_CN = 8
_NCH = _NPW // _CN


def _knn_body(xt_ref, xat_ref, out_ref):
    b = pl.program_id(0)
    xt = xt_ref[0]
    xat = xat_ref[0]
    inner = -2.0 * jnp.dot(xt, xat, preferred_element_type=jnp.float32)
    sq_t = jnp.sum(xt * xt, axis=1, keepdims=True)
    sq_a = jnp.sum(xat * xat, axis=0)
    d = (sq_t + inner) + sq_a[None, :]
    iota = lax.broadcasted_iota(jnp.int32, (_KT, N), 1)
    inf = jnp.float32(jnp.inf)
    cols = []
    for _ in range(KNN):
        m = jnp.min(d, axis=1, keepdims=True)
        idx = jnp.min(jnp.where(d == m, iota, N), axis=1)
        cols.append(idx)
        d = jnp.where(iota == idx[:, None], inf, d)
    out_ref[0] = jnp.stack(cols, axis=1) + b * N


def _knn(xp, xpt):
    return pl.pallas_call(
        _knn_body,
        grid=(B, N // _KT),
        in_specs=[
            pl.BlockSpec((1, _KT, 8), lambda b, t: (b, t, 0)),
            pl.BlockSpec((1, 8, N), lambda b, t: (b, 0, 0)),
        ],
        out_specs=pl.BlockSpec((1, _KT, KNN), lambda b, t: (b, t, 0)),
        out_shape=jax.ShapeDtypeStruct((B, N, KNN), jnp.int32),
    )(xp, xpt)


def _dual_mm_body(x_ref, w_ref, b_ref, y_ref):
    y_ref[...] = jnp.dot(x_ref[...], w_ref[...],
                         preferred_element_type=jnp.float32) + b_ref[...]


def _dual_mm(x, w, bias):
    c = x.shape[1]
    return pl.pallas_call(
        _dual_mm_body,
        grid=(R // _MT,),
        in_specs=[
            pl.BlockSpec((_MT, c), lambda i: (i, 0)),
            pl.BlockSpec((c, 2 * D), lambda i: (0, 0)),
            pl.BlockSpec((1, 2 * D), lambda i: (0, 0)),
        ],
        out_specs=pl.BlockSpec((_MT, 2 * D), lambda i: (i, 0)),
        out_shape=jax.ShapeDtypeStruct((R, 2 * D), jnp.float32),
    )(x, w, bias)


def _gather_max(y, xres, gidx):
    mesh = plsc.VectorSubcoreMesh(core_axis_name="c", subcore_axis_name="s")

    @functools.partial(
        pl.kernel,
        mesh=mesh,
        out_type=jax.ShapeDtypeStruct((R, D), jnp.float32),
        scratch_types=[
            pltpu.VMEM((_CN * KNN,), jnp.int32),
            pltpu.VMEM((_CN * KNN, 2 * D), jnp.float32),
            pltpu.VMEM((_CN, 2 * D), jnp.float32),
            pltpu.VMEM((_CN, D), jnp.float32),
            pltpu.VMEM((_CN, D), jnp.float32),
            pltpu.SemaphoreType.DMA,
        ],
    )
    def k(y_hbm, x_hbm, idx_hbm, out_hbm, idx_v, rows_v, p_v, x_v, o_v, sem):
        wid = lax.axis_index("s") * 2 + lax.axis_index("c")
        base = wid * _NPW

        def body(ci, carry):
            node0 = base + ci * _CN
            pltpu.sync_copy(idx_hbm.at[pl.ds(node0 * KNN, _CN * KNN)], idx_v)
            pltpu.async_copy(y_hbm.at[idx_v], rows_v, sem).wait()
            pltpu.sync_copy(y_hbm.at[pl.ds(node0, _CN)], p_v)
            pltpu.sync_copy(x_hbm.at[pl.ds(node0, _CN)], x_v)
            for i in range(_CN):
                for c in range(D // 16):
                    sl = pl.ds(c * 16, 16)
                    m = rows_v[i * KNN, sl]
                    for r in range(1, KNN):
                        m = jnp.maximum(m, rows_v[i * KNN + r, sl])
                    o = jnp.maximum(p_v[i, pl.ds(D + c * 16, 16)] + m, 0.0)
                    o_v[i, sl] = o + x_v[i, sl]
            pltpu.sync_copy(o_v, out_hbm.at[pl.ds(node0, _CN)])
            return carry

        lax.fori_loop(0, _NCH, body, 0)

    return k(y, xres, gidx)


def _fusion_body(f_ref, w_ref, b_ref, o_ref):
    t = pl.program_id(1)
    y = jnp.dot(f_ref[...], w_ref[...], preferred_element_type=jnp.float32) + b_ref[...]
    y = jnp.maximum(y, 0.0)
    part = jnp.broadcast_to(jnp.max(y, axis=0, keepdims=True), (8, 1024))

    @pl.when(t == 0)
    def _():
        o_ref[0] = part

    @pl.when(t != 0)
    def _():
        o_ref[0] = jnp.maximum(o_ref[0], part)


def _fusion(feats, wt, bias):
    return pl.pallas_call(
        _fusion_body,
        grid=(B, N // _MT),
        in_specs=[
            pl.BlockSpec((_MT, 7 * D), lambda b, t: (b * (N // _MT) + t, 0)),
            pl.BlockSpec((7 * D, 1024), lambda b, t: (0, 0)),
            pl.BlockSpec((1, 1024), lambda b, t: (0, 0)),
        ],
        out_specs=pl.BlockSpec((1, 8, 1024), lambda b, t: (b, 0, 0)),
        out_shape=jax.ShapeDtypeStruct((B, 8, 1024), jnp.float32),
    )(feats, wt, bias)


def _pts_body(f_ref, fu_ref, wa_ref, wb_ref, b1_ref, w2_ref, b2_ref, w3_ref, b3_ref, o_ref):
    z = jnp.dot(fu_ref[0, 0:1, :], wa_ref[...], preferred_element_type=jnp.float32)
    h1 = jnp.dot(f_ref[...], wb_ref[...], preferred_element_type=jnp.float32) + z + b1_ref[...]
    h1 = jnp.maximum(h1, 0.0)
    h2 = jnp.dot(h1, w2_ref[...], preferred_element_type=jnp.float32) + b2_ref[...]
    h2 = jnp.maximum(h2, 0.0)
    h3 = lax.dot_general(w3_ref[...], h2, (((1,), (1,)), ((), ())),
                         preferred_element_type=jnp.float32)
    o_ref[0] = h3 + b3_ref[...][:, 0:1]


def _pts(feats, fmax, wat, wbt, b1, w2t, b2, w3, b3):
    nt = N // _MT
    return pl.pallas_call(
        _pts_body,
        grid=(B, nt),
        in_specs=[
            pl.BlockSpec((_MT, 7 * D), lambda b, t: (b * nt + t, 0)),
            pl.BlockSpec((1, 8, 1024), lambda b, t: (b, 0, 0)),
            pl.BlockSpec((1024, 512), lambda b, t: (0, 0)),
            pl.BlockSpec((7 * D, 512), lambda b, t: (0, 0)),
            pl.BlockSpec((1, 512), lambda b, t: (0, 0)),
            pl.BlockSpec((512, 256), lambda b, t: (0, 0)),
            pl.BlockSpec((1, 256), lambda b, t: (0, 0)),
            pl.BlockSpec((16, 256), lambda b, t: (0, 0)),
            pl.BlockSpec((16, 128), lambda b, t: (0, 0)),
        ],
        out_specs=pl.BlockSpec((1, 16, _MT), lambda b, t: (b, 0, t)),
        out_shape=jax.ShapeDtypeStruct((B, 16, N), jnp.float32),
    )(feats, fmax, wat, wbt, b1, w2t, b2, w3, b3)


def kernel(inputs, W_head, b_head, g_head, be_head, Wb, bb, gb, beb,
           W_f, b_f, g_f, be_f, W_p1, b_p1, g_p1, be_p1,
           W_p2, b_p2, g_p2, be_p2, W_p3, b_p3):
    rsq = jnp.float32(1.0) / jnp.sqrt(jnp.float32(1.0 + BN_EPS))

    x6 = jnp.transpose(inputs[:, :6, :, 0], (0, 2, 1))
    x3p = jnp.pad(x6[:, :, :3], ((0, 0), (0, 0), (0, 5)))
    x3pt = jnp.transpose(x3p, (0, 2, 1))
    gidx = _knn(x3p, x3pt).reshape(R * KNN)

    s = g_head * rsq
    wc = W_head * s[:, None]
    bias0 = (b_head * s + be_head)[None]
    w2 = wc[:, 6:]
    wa = wc[:, :6] - w2
    x0 = jnp.pad(x6.reshape(R, 6), ((0, 0), (0, 2)))
    w0 = jnp.pad(jnp.concatenate([w2, wa], axis=0).T, ((0, 2), (0, 0)))
    y0 = _dual_mm(x0, w0, jnp.concatenate([jnp.zeros_like(bias0), bias0], axis=1))
    feat = _gather_max(y0, jnp.zeros((R, D), jnp.float32), gidx)

    feats = [feat]
    for i in range(6):
        si = gb[i] * rsq
        wci = Wb[i] * si[:, None]
        biasi = (bb[i] * si + beb[i])[None]
        w2i = wci[:, D:]
        wai = wci[:, :D] - w2i
        yi = _dual_mm(feat, jnp.concatenate([w2i, wai], axis=0).T,
                      jnp.concatenate([jnp.zeros_like(biasi), biasi], axis=1))
        feat = _gather_max(yi, feat, gidx)
        feats.append(feat)
    featsc = jnp.concatenate(feats, axis=1)

    sf = g_f * rsq
    fmax = _fusion(featsc, (W_f * sf[:, None]).T, (b_f * sf + be_f)[None])

    s1 = g_p1 * rsq
    w1s = W_p1 * s1[:, None]
    b1 = (b_p1 * s1 + be_p1)[None]
    s2 = g_p2 * rsq
    w2t = (W_p2 * s2[:, None]).T
    b2 = (b_p2 * s2 + be_p2)[None]
    w3 = jnp.pad(W_p3, ((0, 3), (0, 0)))
    b3 = jnp.broadcast_to(jnp.pad(b_p3, (0, 3))[:, None], (16, 128))
    out16 = _pts(featsc, fmax, w1s[:, :1024].T, w1s[:, 1024:].T,
                 b1, w2t, b2, w3, b3)
    return out16[:, :13, :]


# speedup vs baseline: 39.1206x; 1.5281x over previous
import functools

import jax
import jax.numpy as jnp
from jax import lax
from jax.experimental import pallas as pl
from jax.experimental.pallas import tpu as pltpu
from jax.experimental.pallas import tpu_sc as plsc

B = 4
N = 4096
KNN = 16
R = B * N
D = 64
BN_EPS = 1e-5

_KT = 256
_MT = 512
_NW = 32
_NPW = R // _NW
_CN = 16
_NCH = _NPW // _CN


def _knn_body(xt_ref, xat_ref, out_ref):
    b = pl.program_id(0)
    xt = xt_ref[0]
    xat = xat_ref[0]
    inner = -2.0 * jnp.dot(xt, xat, preferred_element_type=jnp.float32)
    sq_t = jnp.sum(xt * xt, axis=1, keepdims=True)
    sq_a = jnp.sum(xat * xat, axis=0)
    d = (sq_t + inner) + sq_a[None, :]
    iota = lax.broadcasted_iota(jnp.int32, (_KT, N), 1)
    inf = jnp.float32(jnp.inf)
    cols = []
    for _ in range(KNN):
        idx = jnp.argmin(d, axis=1).astype(jnp.int32)
        cols.append(idx)
        d = jnp.where(iota == idx[:, None], inf, d)
    out_ref[0] = jnp.stack(cols, axis=1) + b * N


def _knn(xp, xpt):
    return pl.pallas_call(
        _knn_body,
        grid=(B, N // _KT),
        in_specs=[
            pl.BlockSpec((1, _KT, 8), lambda b, t: (b, t, 0)),
            pl.BlockSpec((1, 8, N), lambda b, t: (b, 0, 0)),
        ],
        out_specs=pl.BlockSpec((1, _KT, KNN), lambda b, t: (b, t, 0)),
        out_shape=jax.ShapeDtypeStruct((B, N, KNN), jnp.int32),
    )(xp, xpt)


def _dual_mm_body(x_ref, w_ref, b_ref, y_ref):
    y_ref[...] = jnp.dot(x_ref[...], w_ref[...],
                         preferred_element_type=jnp.float32) + b_ref[...]


def _dual_mm(x, w, bias):
    c = x.shape[1]
    return pl.pallas_call(
        _dual_mm_body,
        grid=(R // _MT,),
        in_specs=[
            pl.BlockSpec((_MT, c), lambda i: (i, 0)),
            pl.BlockSpec((c, 2 * D), lambda i: (0, 0)),
            pl.BlockSpec((1, 2 * D), lambda i: (0, 0)),
        ],
        out_specs=pl.BlockSpec((_MT, 2 * D), lambda i: (i, 0)),
        out_shape=jax.ShapeDtypeStruct((R, 2 * D), jnp.float32),
    )(x, w, bias)


def _gather_max(y, xres, gidx):
    mesh = plsc.VectorSubcoreMesh(core_axis_name="c", subcore_axis_name="s")

    @functools.partial(
        pl.kernel,
        mesh=mesh,
        out_type=jax.ShapeDtypeStruct((R, D), jnp.float32),
        scratch_types=[
            pltpu.VMEM((_NPW * KNN,), jnp.int32),
            pltpu.VMEM((2, _CN * KNN, 2 * D), jnp.float32),
            pltpu.VMEM((2, _CN, 2 * D), jnp.float32),
            pltpu.VMEM((2, _CN, D), jnp.float32),
            pltpu.VMEM((2, _CN, D), jnp.float32),
            pltpu.SemaphoreType.DMA,
            pltpu.SemaphoreType.DMA,
            pltpu.SemaphoreType.DMA,
            pltpu.SemaphoreType.DMA,
        ],
    )
    def k(y_hbm, x_hbm, idx_hbm, out_hbm, idx_all, rows_v, p_v, x_v, o_v,
          gsem0, gsem1, osem0, osem1):
        gsems = (gsem0, gsem1)
        osems = (osem0, osem1)
        wid = lax.axis_index("s") * 2 + lax.axis_index("c")
        base = wid * _NPW
        pltpu.sync_copy(idx_hbm.at[pl.ds(base * KNN, _NPW * KNN)], idx_all)

        def in_copies(ci, buf):
            node0 = base + ci * _CN
            return (
                pltpu.make_async_copy(
                    y_hbm.at[idx_all.at[pl.ds(ci * _CN * KNN, _CN * KNN)]],
                    rows_v.at[buf], gsems[buf]),
                pltpu.make_async_copy(
                    y_hbm.at[pl.ds(node0, _CN)], p_v.at[buf], gsems[buf]),
                pltpu.make_async_copy(
                    x_hbm.at[pl.ds(node0, _CN)], x_v.at[buf], gsems[buf]),
            )

        def out_copy(ci, buf):
            node0 = base + ci * _CN
            return pltpu.make_async_copy(
                o_v.at[buf], out_hbm.at[pl.ds(node0, _CN)], osems[buf])

        for c in in_copies(0, 0) + in_copies(1, 1):
            c.start()

        def chunk(ci, buf):
            for c in in_copies(ci, buf):
                c.wait()

            @pl.when(ci >= 2)
            def _():
                out_copy(ci - 2, buf).wait()

            for i in range(_CN):
                for c in range(D // 16):
                    sl = pl.ds(c * 16, 16)
                    m = rows_v[buf, i * KNN, sl]
                    for r in range(1, KNN):
                        m = jnp.maximum(m, rows_v[buf, i * KNN + r, sl])
                    o = jnp.maximum(p_v[buf, i, pl.ds(D + c * 16, 16)] + m, 0.0)
                    o_v[buf, i, sl] = o + x_v[buf, i, sl]
            out_copy(ci, buf).start()

            @pl.when(ci + 2 < _NCH)
            def _():
                for c in in_copies(ci + 2, buf):
                    c.start()

        def pair(pi, carry):
            chunk(pi * 2, 0)
            chunk(pi * 2 + 1, 1)
            return carry

        lax.fori_loop(0, _NCH // 2, pair, 0)
        out_copy(_NCH - 2, 0).wait()
        out_copy(_NCH - 1, 1).wait()

    return k(y, xres, gidx)


def _fusion_body(f_ref, w_ref, b_ref, o_ref):
    t = pl.program_id(1)
    y = jnp.dot(f_ref[...], w_ref[...], preferred_element_type=jnp.float32) + b_ref[...]
    y = jnp.maximum(y, 0.0)
    part = jnp.broadcast_to(jnp.max(y, axis=0, keepdims=True), (8, 1024))

    @pl.when(t == 0)
    def _():
        o_ref[0] = part

    @pl.when(t != 0)
    def _():
        o_ref[0] = jnp.maximum(o_ref[0], part)


def _fusion(feats, wt, bias):
    return pl.pallas_call(
        _fusion_body,
        grid=(B, N // _MT),
        in_specs=[
            pl.BlockSpec((_MT, 7 * D), lambda b, t: (b * (N // _MT) + t, 0)),
            pl.BlockSpec((7 * D, 1024), lambda b, t: (0, 0)),
            pl.BlockSpec((1, 1024), lambda b, t: (0, 0)),
        ],
        out_specs=pl.BlockSpec((1, 8, 1024), lambda b, t: (b, 0, 0)),
        out_shape=jax.ShapeDtypeStruct((B, 8, 1024), jnp.float32),
    )(feats, wt, bias)


def _pts_body(f_ref, fu_ref, wa_ref, wb_ref, b1_ref, w2_ref, b2_ref, w3_ref, b3_ref, o_ref):
    z = jnp.dot(fu_ref[0, 0:1, :], wa_ref[...], preferred_element_type=jnp.float32)
    h1 = jnp.dot(f_ref[...], wb_ref[...], preferred_element_type=jnp.float32) + z + b1_ref[...]
    h1 = jnp.maximum(h1, 0.0)
    h2 = jnp.dot(h1, w2_ref[...], preferred_element_type=jnp.float32) + b2_ref[...]
    h2 = jnp.maximum(h2, 0.0)
    h3 = lax.dot_general(w3_ref[...], h2, (((1,), (1,)), ((), ())),
                         preferred_element_type=jnp.float32)
    o_ref[0] = h3 + b3_ref[...][:, 0:1]


def _pts(feats, fmax, wat, wbt, b1, w2t, b2, w3, b3):
    nt = N // _MT
    return pl.pallas_call(
        _pts_body,
        grid=(B, nt),
        in_specs=[
            pl.BlockSpec((_MT, 7 * D), lambda b, t: (b * nt + t, 0)),
            pl.BlockSpec((1, 8, 1024), lambda b, t: (b, 0, 0)),
            pl.BlockSpec((1024, 512), lambda b, t: (0, 0)),
            pl.BlockSpec((7 * D, 512), lambda b, t: (0, 0)),
            pl.BlockSpec((1, 512), lambda b, t: (0, 0)),
            pl.BlockSpec((512, 256), lambda b, t: (0, 0)),
            pl.BlockSpec((1, 256), lambda b, t: (0, 0)),
            pl.BlockSpec((16, 256), lambda b, t: (0, 0)),
            pl.BlockSpec((16, 128), lambda b, t: (0, 0)),
        ],
        out_specs=pl.BlockSpec((1, 16, _MT), lambda b, t: (b, 0, t)),
        out_shape=jax.ShapeDtypeStruct((B, 16, N), jnp.float32),
    )(feats, fmax, wat, wbt, b1, w2t, b2, w3, b3)


def kernel(inputs, W_head, b_head, g_head, be_head, Wb, bb, gb, beb,
           W_f, b_f, g_f, be_f, W_p1, b_p1, g_p1, be_p1,
           W_p2, b_p2, g_p2, be_p2, W_p3, b_p3):
    rsq = jnp.float32(1.0) / jnp.sqrt(jnp.float32(1.0 + BN_EPS))

    x6 = jnp.transpose(inputs[:, :6, :, 0], (0, 2, 1))
    x3p = jnp.pad(x6[:, :, :3], ((0, 0), (0, 0), (0, 5)))
    x3pt = jnp.transpose(x3p, (0, 2, 1))
    gidx = _knn(x3p, x3pt).reshape(R * KNN)

    s = g_head * rsq
    wc = W_head * s[:, None]
    bias0 = (b_head * s + be_head)[None]
    w2 = wc[:, 6:]
    wa = wc[:, :6] - w2
    x0 = jnp.pad(x6.reshape(R, 6), ((0, 0), (0, 2)))
    w0 = jnp.pad(jnp.concatenate([w2, wa], axis=0).T, ((0, 2), (0, 0)))
    y0 = _dual_mm(x0, w0, jnp.concatenate([jnp.zeros_like(bias0), bias0], axis=1))
    feat = _gather_max(y0, jnp.zeros((R, D), jnp.float32), gidx)

    feats = [feat]
    for i in range(6):
        si = gb[i] * rsq
        wci = Wb[i] * si[:, None]
        biasi = (bb[i] * si + beb[i])[None]
        w2i = wci[:, D:]
        wai = wci[:, :D] - w2i
        yi = _dual_mm(feat, jnp.concatenate([w2i, wai], axis=0).T,
                      jnp.concatenate([jnp.zeros_like(biasi), biasi], axis=1))
        feat = _gather_max(yi, feat, gidx)
        feats.append(feat)
    featsc = jnp.concatenate(feats, axis=1)

    sf = g_f * rsq
    fmax = _fusion(featsc, (W_f * sf[:, None]).T, (b_f * sf + be_f)[None])

    s1 = g_p1 * rsq
    w1s = W_p1 * s1[:, None]
    b1 = (b_p1 * s1 + be_p1)[None]
    s2 = g_p2 * rsq
    w2t = (W_p2 * s2[:, None]).T
    b2 = (b_p2 * s2 + be_p2)[None]
    w3 = jnp.pad(W_p3, ((0, 3), (0, 0)))
    b3 = jnp.broadcast_to(jnp.pad(b_p3, (0, 3))[:, None], (16, 128))
    out16 = _pts(featsc, fmax, w1s[:, :1024].T, w1s[:, 1024:].T,
                 b1, w2t, b2, w3, b3)
    return out16[:, :13, :]
